# baseline (device time: 49780 ns/iter reference)
import jax
import jax.numpy as jnp
from jax import lax
from jax.experimental import pallas as pl
from jax.experimental.pallas import tpu as pltpu

N_DEV = 4
B = 2
SQ = 128
SKV = 128
D = 512
HQ = 16
HL = 4
DH = 64
HD = HL * DH


def kernel(x, Wq, K_ext, V_ext, Wo):
    def body(x_ref, wq_ref, k_ref, v_ref, wo_ref, out_ref,
             wq_comm, wo_comm, k_rot, v_rot,
             wq_send, wq_recv, wo_send, wo_recv):
        my = lax.axis_index("i")
        left = lax.rem(my + N_DEV - 1, N_DEV)
        right = lax.rem(my + 1, N_DEV)

        barrier = pltpu.get_barrier_semaphore()
        for nbr in (left, right):
            pl.semaphore_signal(
                barrier, inc=1,
                device_id=(nbr,), device_id_type=pl.DeviceIdType.MESH,
            )
        pl.semaphore_wait(barrier, 2)

        for j in range(N_DEV):
            @pl.when(my == j)
            def _(j=j):
                if j == 0:
                    k_rot[...] = k_ref[...]
                    v_rot[...] = v_ref[...]
                else:
                    s = HL * j
                    k_rot[:, :, : HQ - s, :] = k_ref[:, :, s:, :]
                    k_rot[:, :, HQ - s :, :] = k_ref[:, :, :s, :]
                    v_rot[:, :, : HQ - s, :] = v_ref[:, :, s:, :]
                    v_rot[:, :, HQ - s :, :] = v_ref[:, :, :s, :]

        row = lax.broadcasted_iota(jnp.int32, (SQ, SKV), 0)
        col = lax.broadcasted_iota(jnp.int32, (SQ, SKV), 1)
        qb = 2 * my + row // 64
        kb = col // 64
        mask = (qb == kb) | (kb == 0) | (lax.rem(qb + kb, 3) == 0)

        x2d = jnp.reshape(x_ref[...], (B * SQ, D))

        def contrib(wq, wo, kstart):
            q = jnp.dot(x2d, wq, preferred_element_type=jnp.float32)
            outs = []
            for b in range(B):
                ctxs = []
                for j in range(HL):
                    p = kstart + j
                    qbh = q[b * SQ:(b + 1) * SQ, j * DH:(j + 1) * DH]
                    k = k_rot[b, :, p, :]
                    v = v_rot[b, :, p, :]
                    s = lax.dot_general(
                        qbh, k, (((1,), (1,)), ((), ())),
                        preferred_element_type=jnp.float32,
                    ) * 0.125
                    s = jnp.where(mask, s, -1e9)
                    m = jnp.max(s, axis=-1, keepdims=True)
                    w = jnp.exp(s - m)
                    w = w / jnp.sum(w, axis=-1, keepdims=True)
                    ctxs.append(
                        jnp.dot(w, v, preferred_element_type=jnp.float32))
                ctx = jnp.concatenate(ctxs, axis=1)
                outs.append(
                    jnp.dot(ctx, wo, preferred_element_type=jnp.float32))
            return outs

        acc = None
        pending = []
        for h in range(N_DEV - 1):
            src_wq = wq_ref if h == 0 else wq_comm.at[h - 1]
            src_wo = wo_ref if h == 0 else wo_comm.at[h - 1]
            r_wq = pltpu.make_async_remote_copy(
                src_ref=src_wq, dst_ref=wq_comm.at[h],
                send_sem=wq_send.at[h], recv_sem=wq_recv.at[h],
                device_id=(right,), device_id_type=pl.DeviceIdType.MESH,
            )
            r_wo = pltpu.make_async_remote_copy(
                src_ref=src_wo, dst_ref=wo_comm.at[h],
                send_sem=wo_send.at[h], recv_sem=wo_recv.at[h],
                device_id=(right,), device_id_type=pl.DeviceIdType.MESH,
            )
            r_wq.start()
            r_wo.start()
            pending += [r_wq, r_wo]

            wq_v = wq_ref[...] if h == 0 else wq_comm[h - 1]
            wo_v = wo_ref[...] if h == 0 else wo_comm[h - 1]
            c = contrib(wq_v, wo_v, (HQ - HL * h) % HQ)
            acc = c if acc is None else [a + d for a, d in zip(acc, c)]

            r_wq.wait_recv()
            r_wo.wait_recv()

        c = contrib(wq_comm[N_DEV - 2], wo_comm[N_DEV - 2],
                    (HQ - HL * (N_DEV - 1)) % HQ)
        acc = [a + d for a, d in zip(acc, c)]
        for b in range(B):
            out_ref[b, :, :] = acc[b]

        for r in pending:
            r.wait_send()

    return pl.pallas_call(
        body,
        out_shape=jax.ShapeDtypeStruct((B, SQ, D), jnp.float32),
        in_specs=[pl.BlockSpec(memory_space=pltpu.VMEM)] * 5,
        out_specs=pl.BlockSpec(memory_space=pltpu.VMEM),
        scratch_shapes=[
            pltpu.VMEM((N_DEV - 1, D, HD), jnp.float32),
            pltpu.VMEM((N_DEV - 1, HD, D), jnp.float32),
            pltpu.VMEM((B, SKV, HQ, DH), jnp.float32),
            pltpu.VMEM((B, SKV, HQ, DH), jnp.float32),
            pltpu.SemaphoreType.DMA((N_DEV - 1,)),
            pltpu.SemaphoreType.DMA((N_DEV - 1,)),
            pltpu.SemaphoreType.DMA((N_DEV - 1,)),
            pltpu.SemaphoreType.DMA((N_DEV - 1,)),
        ],
        compiler_params=pltpu.CompilerParams(collective_id=0),
    )(x, Wq, K_ext, V_ext, Wo)


# device time: 21289 ns/iter; 2.3383x vs baseline; 2.3383x over previous
import jax
import jax.numpy as jnp
from jax import lax
from jax.experimental import pallas as pl
from jax.experimental.pallas import tpu as pltpu

N_DEV = 4
B = 2
SQ = 128
SKV = 128
D = 512
HQ = 16
HL = 4
DH = 64
HD = HL * DH
HH = HD // 2
CDT = jnp.bfloat16
COMM_DT = jnp.float32

MINE, FROM_L, FROM_R, DIAG = 0, 1, 2, 3


def kernel(x, Wq, K_ext, V_ext, Wo):
    def body(x_ref, wq_ref, k_ref, v_ref, wo_ref, out_ref,
             wqA, woA, wqB, woB, k_rot, v_rot, send_sems, recv_sems):
        my = lax.axis_index("i")
        left = lax.rem(my + N_DEV - 1, N_DEV)
        right = lax.rem(my + 1, N_DEV)

        barrier = pltpu.get_barrier_semaphore()
        for nbr in (left, right):
            pl.semaphore_signal(
                barrier, inc=1,
                device_id=(nbr,), device_id_type=pl.DeviceIdType.MESH,
            )
        pl.semaphore_wait(barrier, 2)

        wq_v = wq_ref[...].astype(CDT)
        wo_v = wo_ref[...].astype(CDT)
        wqA[MINE] = pltpu.bitcast(wq_v[:, :HH], COMM_DT)
        wqB[MINE] = pltpu.bitcast(wq_v[:, HH:], COMM_DT)
        woA[MINE] = pltpu.bitcast(wo_v[:HH, :], COMM_DT)
        woB[MINE] = pltpu.bitcast(wo_v[HH:, :], COMM_DT)

        sem_ix = iter(range(12))
        pending = []

        def rdma(buf, src_slot, dst_slot, dst):
            i = next(sem_ix)
            r = pltpu.make_async_remote_copy(
                src_ref=buf.at[src_slot], dst_ref=buf.at[dst_slot],
                send_sem=send_sems.at[i], recv_sem=recv_sems.at[i],
                device_id=(dst,), device_id_type=pl.DeviceIdType.MESH,
            )
            r.start()
            pending.append(r)
            return r

        r1_wqA_R = rdma(wqA, MINE, FROM_L, right)
        r1_woA_R = rdma(woA, MINE, FROM_L, right)
        r1_wqB_L = rdma(wqB, MINE, FROM_R, left)
        r1_woB_L = rdma(woB, MINE, FROM_R, left)
        r1_wqB_R = rdma(wqB, MINE, FROM_L, right)
        r1_woB_R = rdma(woB, MINE, FROM_L, right)
        r1_wqA_L = rdma(wqA, MINE, FROM_R, left)
        r1_woA_L = rdma(woA, MINE, FROM_R, left)

        for j in range(N_DEV):
            @pl.when(my == j)
            def _(j=j):
                if j == 0:
                    k_rot[...] = k_ref[...]
                    v_rot[...] = v_ref[...]
                else:
                    s = HL * j
                    k_rot[:, :, : HQ - s, :] = k_ref[:, :, s:, :]
                    k_rot[:, :, HQ - s :, :] = k_ref[:, :, :s, :]
                    v_rot[:, :, : HQ - s, :] = v_ref[:, :, s:, :]
                    v_rot[:, :, HQ - s :, :] = v_ref[:, :, :s, :]

        row = lax.broadcasted_iota(jnp.int32, (SQ, SKV), 0)
        col = lax.broadcasted_iota(jnp.int32, (SQ, SKV), 1)
        qb = 2 * my + row // 64
        kb = col // 64
        mask = (qb == kb) | (kb == 0) | (lax.rem(qb + kb, 3) == 0)

        x2d = jnp.reshape(x_ref[...], (B * SQ, D)).astype(CDT)

        def contrib(wq_h, wo_h, p0):
            q = jnp.dot(x2d, wq_h, preferred_element_type=jnp.float32)
            outs = []
            for b in range(B):
                ctxs = []
                for j in range(2):
                    p = p0 + j
                    qbh = q[b * SQ:(b + 1) * SQ, j * DH:(j + 1) * DH]
                    qbh = qbh.astype(CDT)
                    k = k_rot[b, :, p, :].astype(CDT)
                    v = v_rot[b, :, p, :].astype(CDT)
                    s = lax.dot_general(
                        qbh, k, (((1,), (1,)), ((), ())),
                        preferred_element_type=jnp.float32,
                    ) * 0.125
                    s = jnp.where(mask, s, -1e9)
                    m = jnp.max(s, axis=-1, keepdims=True)
                    w = jnp.exp(s - m)
                    w = (w / jnp.sum(w, axis=-1, keepdims=True)).astype(CDT)
                    ctxs.append(
                        jnp.dot(w, v, preferred_element_type=jnp.float32))
                ctx = jnp.concatenate(ctxs, axis=1).astype(CDT)
                outs.append(
                    jnp.dot(ctx, wo_h, preferred_element_type=jnp.float32))
            return outs

        def pair(buf_wq, buf_wo, slot, base):
            return contrib(pltpu.bitcast(buf_wq[slot], CDT),
                           pltpu.bitcast(buf_wo[slot], CDT), base)

        def add(acc, c):
            return [a + d for a, d in zip(acc, c)]

        acc = pair(wqA, woA, MINE, 0)
        acc = add(acc, pair(wqB, woB, MINE, 2))

        r1_wqA_R.wait_recv()
        r1_woA_R.wait_recv()
        h2_A = [rdma(wqA, FROM_L, DIAG, right),
                rdma(woA, FROM_L, DIAG, right)]
        r1_wqB_L.wait_recv()
        r1_woB_L.wait_recv()
        h2_B = [rdma(wqB, FROM_R, DIAG, left),
                rdma(woB, FROM_R, DIAG, left)]

        acc = add(acc, pair(wqA, woA, FROM_L, HQ - HL))
        acc = add(acc, pair(wqB, woB, FROM_R, HL + 2))
        r1_wqB_R.wait_recv()
        r1_woB_R.wait_recv()
        acc = add(acc, pair(wqB, woB, FROM_L, HQ - HL + 2))
        r1_wqA_L.wait_recv()
        r1_woA_L.wait_recv()
        acc = add(acc, pair(wqA, woA, FROM_R, HL))

        for r in h2_A + h2_B:
            r.wait_recv()
        acc = add(acc, pair(wqA, woA, DIAG, 2 * HL))
        acc = add(acc, pair(wqB, woB, DIAG, 2 * HL + 2))
        for b in range(B):
            out_ref[b, :, :] = acc[b]

        for r in pending:
            r.wait_send()

    return pl.pallas_call(
        body,
        out_shape=jax.ShapeDtypeStruct((B, SQ, D), jnp.float32),
        in_specs=[pl.BlockSpec(memory_space=pltpu.VMEM)] * 5,
        out_specs=pl.BlockSpec(memory_space=pltpu.VMEM),
        scratch_shapes=[
            pltpu.VMEM((4, D // 2, HH), COMM_DT),
            pltpu.VMEM((4, HH // 2, D), COMM_DT),
            pltpu.VMEM((4, D // 2, HH), COMM_DT),
            pltpu.VMEM((4, HH // 2, D), COMM_DT),
            pltpu.VMEM((B, SKV, HQ, DH), COMM_DT),
            pltpu.VMEM((B, SKV, HQ, DH), COMM_DT),
            pltpu.SemaphoreType.DMA((12,)),
            pltpu.SemaphoreType.DMA((12,)),
        ],
        compiler_params=pltpu.CompilerParams(collective_id=0),
    )(x, Wq, K_ext, V_ext, Wo)


# device time: 20360 ns/iter; 2.4450x vs baseline; 1.0456x over previous
import jax
import jax.numpy as jnp
from jax import lax
from jax.experimental import pallas as pl
from jax.experimental.pallas import tpu as pltpu

N_DEV = 4
B = 2
SQ = 128
SKV = 128
D = 512
HQ = 16
HL = 4
DH = 64
HD = HL * DH
HH = HD // 2
CDT = jnp.bfloat16
COMM_DT = jnp.float32

MINE, FROM_L, FROM_R, DIAG = 0, 1, 2, 3


def kernel(x, Wq, K_ext, V_ext, Wo):
    def body(x_ref, wq_ref, k_ref, v_ref, wo_ref, out_ref,
             wqA, woA, wqB, woB, k_rot, v_rot, send_sems, recv_sems):
        my = lax.axis_index("i")
        left = lax.rem(my + N_DEV - 1, N_DEV)
        right = lax.rem(my + 1, N_DEV)

        barrier = pltpu.get_barrier_semaphore()
        for nbr in (left, right):
            pl.semaphore_signal(
                barrier, inc=1,
                device_id=(nbr,), device_id_type=pl.DeviceIdType.MESH,
            )
        pl.semaphore_wait(barrier, 2)

        wq_v = wq_ref[...].astype(CDT)
        wo_v = wo_ref[...].astype(CDT)
        wqA[MINE] = pltpu.bitcast(wq_v[:, :HH], COMM_DT)
        wqB[MINE] = pltpu.bitcast(wq_v[:, HH:], COMM_DT)
        woA[MINE] = pltpu.bitcast(wo_v[:HH, :], COMM_DT)
        woB[MINE] = pltpu.bitcast(wo_v[HH:, :], COMM_DT)

        sem_ix = iter(range(12))
        pending = []

        def rdma(buf, src_slot, dst_slot, dst):
            i = next(sem_ix)
            r = pltpu.make_async_remote_copy(
                src_ref=buf.at[src_slot], dst_ref=buf.at[dst_slot],
                send_sem=send_sems.at[i], recv_sem=recv_sems.at[i],
                device_id=(dst,), device_id_type=pl.DeviceIdType.MESH,
            )
            r.start()
            pending.append(r)
            return r

        r1_wqA_R = rdma(wqA, MINE, FROM_L, right)
        r1_woA_R = rdma(woA, MINE, FROM_L, right)
        r1_wqB_L = rdma(wqB, MINE, FROM_R, left)
        r1_woB_L = rdma(woB, MINE, FROM_R, left)
        r1_wqB_R = rdma(wqB, MINE, FROM_L, right)
        r1_woB_R = rdma(woB, MINE, FROM_L, right)
        r1_wqA_L = rdma(wqA, MINE, FROM_R, left)
        r1_woA_L = rdma(woA, MINE, FROM_R, left)

        for j in range(N_DEV):
            @pl.when(my == j)
            def _(j=j):
                if j == 0:
                    k_rot[...] = k_ref[...]
                    v_rot[...] = v_ref[...]
                else:
                    s = HL * j
                    k_rot[:, :, : HQ - s, :] = k_ref[:, :, s:, :]
                    k_rot[:, :, HQ - s :, :] = k_ref[:, :, :s, :]
                    v_rot[:, :, : HQ - s, :] = v_ref[:, :, s:, :]
                    v_rot[:, :, HQ - s :, :] = v_ref[:, :, :s, :]

        row = lax.broadcasted_iota(jnp.int32, (SQ, SKV), 0)
        col = lax.broadcasted_iota(jnp.int32, (SQ, SKV), 1)
        qb = 2 * my + row // 64
        kb = col // 64
        mask = (qb == kb) | (kb == 0) | (lax.rem(qb + kb, 3) == 0)

        x2d = jnp.reshape(x_ref[...], (B * SQ, D)).astype(CDT)
        mask4 = jnp.concatenate([mask] * (2 * B), axis=0)

        def contrib(wq_h, wo_h, p0):
            q = jnp.dot(x2d, wq_h, preferred_element_type=jnp.float32)
            blocks = []
            for b in range(B):
                for j in range(2):
                    qbh = q[b * SQ:(b + 1) * SQ, j * DH:(j + 1) * DH]
                    qbh = qbh.astype(CDT)
                    k = k_rot[b, :, p0 + j, :].astype(CDT)
                    blocks.append(lax.dot_general(
                        qbh, k, (((1,), (1,)), ((), ())),
                        preferred_element_type=jnp.float32,
                    ))
            s = jnp.concatenate(blocks, axis=0) * 0.125
            s = jnp.where(mask4, s, -1e9)
            m = jnp.max(s, axis=-1, keepdims=True)
            w = jnp.exp(s - m)
            w = (w / jnp.sum(w, axis=-1, keepdims=True)).astype(CDT)
            outs = []
            for b in range(B):
                ctxs = []
                for j in range(2):
                    wb = w[(2 * b + j) * SQ:(2 * b + j + 1) * SQ, :]
                    v = v_rot[b, :, p0 + j, :].astype(CDT)
                    ctxs.append(
                        jnp.dot(wb, v, preferred_element_type=jnp.float32))
                ctx = jnp.concatenate(ctxs, axis=1).astype(CDT)
                outs.append(
                    jnp.dot(ctx, wo_h, preferred_element_type=jnp.float32))
            return outs

        def pair(buf_wq, buf_wo, slot, base):
            return contrib(pltpu.bitcast(buf_wq[slot], CDT),
                           pltpu.bitcast(buf_wo[slot], CDT), base)

        def add(acc, c):
            return [a + d for a, d in zip(acc, c)]

        acc = pair(wqA, woA, MINE, 0)
        acc = add(acc, pair(wqB, woB, MINE, 2))

        r1_wqA_R.wait_recv()
        h2_A = [rdma(wqA, FROM_L, DIAG, right)]
        r1_woA_R.wait_recv()
        h2_A.append(rdma(woA, FROM_L, DIAG, right))
        r1_wqB_L.wait_recv()
        h2_B = [rdma(wqB, FROM_R, DIAG, left)]
        r1_woB_L.wait_recv()
        h2_B.append(rdma(woB, FROM_R, DIAG, left))

        acc = add(acc, pair(wqA, woA, FROM_L, HQ - HL))
        acc = add(acc, pair(wqB, woB, FROM_R, HL + 2))
        r1_wqB_R.wait_recv()
        r1_woB_R.wait_recv()
        acc = add(acc, pair(wqB, woB, FROM_L, HQ - HL + 2))
        r1_wqA_L.wait_recv()
        r1_woA_L.wait_recv()
        acc = add(acc, pair(wqA, woA, FROM_R, HL))

        for r in h2_A:
            r.wait_recv()
        acc = add(acc, pair(wqA, woA, DIAG, 2 * HL))
        for r in h2_B:
            r.wait_recv()
        acc = add(acc, pair(wqB, woB, DIAG, 2 * HL + 2))
        for b in range(B):
            out_ref[b, :, :] = acc[b]

        for r in pending:
            r.wait_send()

    return pl.pallas_call(
        body,
        out_shape=jax.ShapeDtypeStruct((B, SQ, D), jnp.float32),
        in_specs=[pl.BlockSpec(memory_space=pltpu.VMEM)] * 5,
        out_specs=pl.BlockSpec(memory_space=pltpu.VMEM),
        scratch_shapes=[
            pltpu.VMEM((4, D // 2, HH), COMM_DT),
            pltpu.VMEM((4, HH // 2, D), COMM_DT),
            pltpu.VMEM((4, D // 2, HH), COMM_DT),
            pltpu.VMEM((4, HH // 2, D), COMM_DT),
            pltpu.VMEM((B, SKV, HQ, DH), COMM_DT),
            pltpu.VMEM((B, SKV, HQ, DH), COMM_DT),
            pltpu.SemaphoreType.DMA((12,)),
            pltpu.SemaphoreType.DMA((12,)),
        ],
        compiler_params=pltpu.CompilerParams(collective_id=0),
    )(x, Wq, K_ext, V_ext, Wo)
